# Initial kernel scaffold; baseline (speedup 1.0000x reference)
import jax
import jax.numpy as jnp
from jax.experimental import pallas as pl


def kernel(adj1, adj2, W0, b0, Wl, bl, Wr, Wl_last, bl_last, Wr_last, mW1, mb1, mW2, mb2):
    raise NotImplementedError("write your pallas kernel here")


# speedup vs baseline: 5.5252x; 5.5252x over previous
import functools

import jax
import jax.numpy as jnp
from jax import lax
from jax.experimental import pallas as pl
from jax.experimental.pallas import tpu as pltpu
from jax.experimental.pallas import tpu_sc as plsc

N = 10000
D = 128
E = 320000

NC = 2
NS = 16
NW = NC * NS
CH = 80
EPW = E // NW
ROWS_PW = EPW // CH
NP = 10240
NPW = NP // NS

_mesh = plsc.VectorSubcoreMesh(core_axis_name="c", subcore_axis_name="s")


def _segsum_body(x_hbm, src_hbm, dst_hbm, z_hbm, out_hbm,
                 src_v, dst_v, rows_v, acc_sh, sem):
    c = lax.axis_index("c")
    s = lax.axis_index("s")
    wid = s * NC + c

    pltpu.sync_copy(z_hbm.at[pl.ds(s * NPW, NPW)],
                    acc_sh.at[pl.ds(s * NPW, NPW)])
    pltpu.sync_copy(src_hbm.at[pl.ds(wid * EPW, EPW)], src_v)
    pltpu.sync_copy(dst_hbm.at[wid], dst_v)
    plsc.subcore_barrier()

    def body(j, carry):
        pltpu.async_copy(x_hbm.at[src_v.at[pl.ds(j * CH, CH)]],
                         rows_v, sem).wait()
        pltpu.sync_copy(rows_v, acc_sh.at[dst_v.at[j]], add=True)
        return carry

    lax.fori_loop(0, ROWS_PW, body, 0, unroll=False)
    plsc.subcore_barrier()
    pltpu.sync_copy(acc_sh.at[pl.ds(s * NPW, NPW)],
                    out_hbm.at[c, pl.ds(s * NPW, NPW)])


_segsum = pl.kernel(
    _segsum_body,
    out_type=jax.ShapeDtypeStruct((NC, NP, D), jnp.float32),
    mesh=_mesh,
    scratch_types=[
        pltpu.VMEM((EPW,), jnp.int32),
        pltpu.VMEM((ROWS_PW, CH), jnp.int32),
        pltpu.VMEM((CH, D), jnp.float32),
        pltpu.VMEM_SHARED((NP, D), jnp.float32),
        pltpu.SemaphoreType.DMA,
    ],
)


def _segcnt_body(dst_hbm, z_hbm, out_hbm, dst_v, ones_v, acc_sh, sem):
    c = lax.axis_index("c")
    s = lax.axis_index("s")
    wid = s * NC + c

    pltpu.sync_copy(z_hbm.at[pl.ds(s * NPW, NPW)],
                    acc_sh.at[pl.ds(s * NPW, NPW)])
    pltpu.sync_copy(dst_hbm.at[wid], dst_v)
    one16 = jnp.full((16,), 1.0, jnp.float32)
    for r in range(CH):
        for k in range(D // 16):
            ones_v[r, pl.ds(k * 16, 16)] = one16
    plsc.subcore_barrier()

    def body(j, carry):
        pltpu.sync_copy(ones_v, acc_sh.at[dst_v.at[j]], add=True)
        return carry

    lax.fori_loop(0, ROWS_PW, body, 0, unroll=False)
    plsc.subcore_barrier()
    pltpu.sync_copy(acc_sh.at[pl.ds(s * NPW, NPW)],
                    out_hbm.at[c, pl.ds(s * NPW, NPW)])


_segcnt = pl.kernel(
    _segcnt_body,
    out_type=jax.ShapeDtypeStruct((NC, NP, D), jnp.float32),
    mesh=_mesh,
    scratch_types=[
        pltpu.VMEM((ROWS_PW, CH), jnp.int32),
        pltpu.VMEM((CH, D), jnp.float32),
        pltpu.VMEM_SHARED((NP, D), jnp.float32),
        pltpu.SemaphoreType.DMA,
    ],
)


_RB = 400
_GRID = N // _RB


def _init_fn(a_ref, w_ref, b_ref, o_ref):
    h = jnp.dot(a_ref[...], w_ref[...], preferred_element_type=jnp.float32)
    h = jnp.maximum(h + b_ref[...], 0.0)
    n = jnp.sqrt(jnp.sum(h * h, axis=1, keepdims=True))
    o_ref[...] = h / jnp.maximum(n, 1e-12)


def _init_matmul(adj1, W0, b0):
    rb = 200
    return pl.pallas_call(
        _init_fn,
        grid=(N // rb,),
        in_specs=[
            pl.BlockSpec((rb, N), lambda i: (i, 0)),
            pl.BlockSpec((N, D), lambda i: (0, 0)),
            pl.BlockSpec((1, D), lambda i: (0, 0)),
        ],
        out_specs=pl.BlockSpec((rb, D), lambda i: (i, 0)),
        out_shape=jax.ShapeDtypeStruct((N, D), jnp.float32),
    )(adj1, W0, b0.reshape(1, D))


def _mlp_head(h, mw1_ref, mb1_ref, w2_ref, mb2_ref):
    h1 = jnp.dot(h, mw1_ref[...], preferred_element_type=jnp.float32)
    h1 = jnp.maximum(h1 + mb1_ref[...], 0.0)
    s = jnp.sum(h1 * w2_ref[...], axis=1, keepdims=True)
    return s + mb2_ref[...][:, :1]


def _layer_fn(part_ref, cnt_ref, x_ref, wl_ref, bl_ref, wr_ref,
              mw1_ref, mb1_ref, w2_ref, mb2_ref, xo_ref, *outs,
              l2, mlp):
    p3 = part_ref[...]
    c3 = cnt_ref[...]
    cc = c3[0, :, :1] + c3[1, :, :1]
    mean = (p3[0] + p3[1]) / jnp.maximum(cc, 1.0)
    h = (jnp.dot(mean, wl_ref[...], preferred_element_type=jnp.float32)
         + bl_ref[...]
         + jnp.dot(x_ref[...], wr_ref[...], preferred_element_type=jnp.float32))
    h = jnp.maximum(h, 0.0)
    if l2:
        n = jnp.sqrt(jnp.sum(h * h, axis=1, keepdims=True))
        h = h / jnp.maximum(n, 1e-12)
    xo_ref[...] = h
    if mlp:
        outs[0][...] = _mlp_head(h, mw1_ref, mb1_ref, w2_ref, mb2_ref)


def _layer(part, cnt, x, wl, bl, wr, mw1, mb1, w2, mb2, *, l2, mlp):
    outs = [jax.ShapeDtypeStruct((N, D), jnp.float32)]
    out_specs = [pl.BlockSpec((_RB, D), lambda i: (i, 0))]
    if mlp:
        outs.append(jax.ShapeDtypeStruct((N, 1), jnp.float32))
        out_specs.append(pl.BlockSpec((_RB, 1), lambda i: (i, 0)))
    full = lambda shape: pl.BlockSpec(shape, lambda i: tuple(0 for _ in shape))
    res = pl.pallas_call(
        functools.partial(_layer_fn, l2=l2, mlp=mlp),
        grid=(_GRID,),
        in_specs=[
            pl.BlockSpec((NC, _RB, D), lambda i: (0, i, 0)),
            pl.BlockSpec((NC, _RB, D), lambda i: (0, i, 0)),
            pl.BlockSpec((_RB, D), lambda i: (i, 0)),
            full((D, D)), full((1, D)), full((D, D)),
            full((D, D)), full((1, D)), full((1, D)), full((1, D)),
        ],
        out_specs=out_specs,
        out_shape=outs,
    )(part, cnt, x, wl, bl, wr, mw1, mb1, w2, mb2)
    if not mlp:
        return res[0], None
    return res


def _layer_nofuse_so_fn(x_ref, mw1_ref, mb1_ref, w2_ref, mb2_ref, so_ref):
    so_ref[...] = _mlp_head(x_ref[...], mw1_ref, mb1_ref, w2_ref, mb2_ref)


def _mlp_only(x, mw1, mb1, w2, mb2):
    full = lambda shape: pl.BlockSpec(shape, lambda i: tuple(0 for _ in shape))
    return pl.pallas_call(
        _layer_nofuse_so_fn,
        grid=(_GRID,),
        in_specs=[
            pl.BlockSpec((_RB, D), lambda i: (i, 0)),
            full((D, D)), full((1, D)), full((1, D)), full((1, D)),
        ],
        out_specs=pl.BlockSpec((_RB, 1), lambda i: (i, 0)),
        out_shape=jax.ShapeDtypeStruct((N, 1), jnp.float32),
    )(x, mw1, mb1, w2, mb2)


def _sum_fn(p_ref, o_ref):
    o_ref[...] = jnp.sum(p_ref[...], axis=1, keepdims=True)


def _sum_scores(parts):
    p = jnp.concatenate(parts, axis=1)
    k = p.shape[1]
    return pl.pallas_call(
        _sum_fn,
        grid=(_GRID,),
        in_specs=[pl.BlockSpec((_RB, k), lambda i: (i, 0))],
        out_specs=pl.BlockSpec((_RB, 1), lambda i: (i, 0)),
        out_shape=jax.ShapeDtypeStruct((N, 1), jnp.float32),
    )(p)


def kernel(adj1, adj2, W0, b0, Wl, bl, Wr, Wl_last, bl_last, Wr_last,
           mW1, mb1, mW2, mb2):
    src1d = adj2[0]
    dst3d = adj2[1].reshape(NW, ROWS_PW, CH)
    zeros = jnp.zeros((NP, D), jnp.float32)
    w2 = mW2.reshape(1, D)
    mb1r = mb1.reshape(1, D)
    mb2r = jnp.broadcast_to(mb2.reshape(1, 1), (1, D))
    mhead = (mW1, mb1r, w2, mb2r)

    cnt = _segcnt(dst3d, zeros)
    x0 = _init_matmul(adj1, W0, b0)

    m0 = _segsum(x0, src1d, dst3d, zeros)

    score_parts = [_mlp_only(x0, *mhead)]
    x1 = None
    for i in range(Wl.shape[0]):
        xi, si = _layer(m0, cnt, x0, Wl[i], bl[i].reshape(1, D), Wr[i],
                        *mhead, l2=True, mlp=True)
        score_parts.append(si)
        if i == 0:
            x1 = xi

    x = x1
    for i in range(1, Wl.shape[0]):
        mi = _segsum(x, src1d, dst3d, zeros)
        x, _ = _layer(mi, cnt, x, Wl[i], bl[i].reshape(1, D), Wr[i],
                      *mhead, l2=True, mlp=False)

    m_last = _segsum(x, src1d, dst3d, zeros)
    _, s_last = _layer(m_last, cnt, x, Wl_last, bl_last.reshape(1, D),
                       Wr_last, *mhead, l2=False, mlp=True)
    score_parts.append(s_last)

    return _sum_scores(score_parts)


# speedup vs baseline: 7.8708x; 1.4245x over previous
import functools

import jax
import jax.numpy as jnp
from jax import lax
from jax.experimental import pallas as pl
from jax.experimental.pallas import tpu as pltpu
from jax.experimental.pallas import tpu_sc as plsc

N = 10000
D = 128
E = 320000

NC = 2
NS = 16
NW = NC * NS
CH = 80
EPW = E // NW
ROWS_PW = EPW // CH
NP = 10240
NPW = NP // NS

_mesh = plsc.VectorSubcoreMesh(core_axis_name="c", subcore_axis_name="s")


def _segsum_body(x_hbm, src_hbm, dst_hbm, z_hbm, out_hbm,
                 src_v, dst_v, rows0, rows1, acc_sh,
                 sem0, sem1, ssem0, ssem1):
    c = lax.axis_index("c")
    s = lax.axis_index("s")
    wid = s * NC + c

    pltpu.sync_copy(z_hbm.at[pl.ds(s * NPW, NPW)],
                    acc_sh.at[pl.ds(s * NPW, NPW)])
    pltpu.sync_copy(src_hbm.at[pl.ds(wid * EPW, EPW)], src_v)
    pltpu.sync_copy(dst_hbm.at[wid], dst_v)
    plsc.subcore_barrier()

    def gather(j, buf, sem):
        pltpu.async_copy(x_hbm.at[src_v.at[pl.ds(j * CH, CH)]], buf, sem)

    def drain_scatter(j, buf, gsem, ssem):
        pltpu.make_async_copy(x_hbm.at[src_v.at[pl.ds(j * CH, CH)]], buf,
                              gsem).wait()
        pltpu.async_copy(buf, acc_sh.at[dst_v.at[j]], ssem, add=True)
        pltpu.make_async_copy(buf, acc_sh.at[dst_v.at[j]], ssem).wait()

    gather(0, rows0, sem0)

    def body(j, carry):
        gather(2 * j + 1, rows1, sem1)
        drain_scatter(2 * j, rows0, sem0, ssem0)
        gather(2 * j + 2, rows0, sem0)
        drain_scatter(2 * j + 1, rows1, sem1, ssem1)
        return carry

    lax.fori_loop(0, (ROWS_PW - 1) // 2, body, 0, unroll=False)
    drain_scatter(ROWS_PW - 1, rows0, sem0, ssem0)

    plsc.subcore_barrier()
    pltpu.sync_copy(acc_sh.at[pl.ds(s * NPW, NPW)],
                    out_hbm.at[c, pl.ds(s * NPW, NPW)])


_segsum = pl.kernel(
    _segsum_body,
    out_type=jax.ShapeDtypeStruct((NC, NP, D), jnp.float32),
    mesh=_mesh,
    scratch_types=[
        pltpu.VMEM((EPW,), jnp.int32),
        pltpu.VMEM((ROWS_PW, CH), jnp.int32),
        pltpu.VMEM((CH, D), jnp.float32),
        pltpu.VMEM((CH, D), jnp.float32),
        pltpu.VMEM_SHARED((NP, D), jnp.float32),
        pltpu.SemaphoreType.DMA,
        pltpu.SemaphoreType.DMA,
        pltpu.SemaphoreType.DMA,
        pltpu.SemaphoreType.DMA,
    ],
)


def _segcnt_body(dst_hbm, z_hbm, out_hbm, dst_v, ones_v, acc_sh, sem):
    c = lax.axis_index("c")
    s = lax.axis_index("s")
    wid = s * NC + c

    pltpu.sync_copy(z_hbm.at[pl.ds(s * NPW, NPW)],
                    acc_sh.at[pl.ds(s * NPW, NPW)])
    pltpu.sync_copy(dst_hbm.at[wid], dst_v)
    one16 = jnp.full((16,), 1.0, jnp.float32)
    for r in range(CH):
        for k in range(D // 16):
            ones_v[r, pl.ds(k * 16, 16)] = one16
    plsc.subcore_barrier()

    def body(j, carry):
        pltpu.sync_copy(ones_v, acc_sh.at[dst_v.at[j]], add=True)
        return carry

    lax.fori_loop(0, ROWS_PW, body, 0, unroll=False)
    plsc.subcore_barrier()
    pltpu.sync_copy(acc_sh.at[pl.ds(s * NPW, NPW)],
                    out_hbm.at[c, pl.ds(s * NPW, NPW)])


_segcnt = pl.kernel(
    _segcnt_body,
    out_type=jax.ShapeDtypeStruct((NC, NP, D), jnp.float32),
    mesh=_mesh,
    scratch_types=[
        pltpu.VMEM((ROWS_PW, CH), jnp.int32),
        pltpu.VMEM((CH, D), jnp.float32),
        pltpu.VMEM_SHARED((NP, D), jnp.float32),
        pltpu.SemaphoreType.DMA,
    ],
)


_RB = 400
_GRID = N // _RB


def _init_fn(a_ref, w_ref, b_ref, o_ref):
    h = jnp.dot(a_ref[...], w_ref[...], preferred_element_type=jnp.float32)
    h = jnp.maximum(h + b_ref[...], 0.0)
    n = jnp.sqrt(jnp.sum(h * h, axis=1, keepdims=True))
    o_ref[...] = h / jnp.maximum(n, 1e-12)


def _init_matmul(adj1, W0, b0):
    rb = 200
    return pl.pallas_call(
        _init_fn,
        grid=(N // rb,),
        in_specs=[
            pl.BlockSpec((rb, N), lambda i: (i, 0)),
            pl.BlockSpec((N, D), lambda i: (0, 0)),
            pl.BlockSpec((1, D), lambda i: (0, 0)),
        ],
        out_specs=pl.BlockSpec((rb, D), lambda i: (i, 0)),
        out_shape=jax.ShapeDtypeStruct((N, D), jnp.float32),
    )(adj1, W0, b0.reshape(1, D))


def _bdot(a, b):
    return jnp.dot(a.astype(jnp.bfloat16), b.astype(jnp.bfloat16),
                   preferred_element_type=jnp.float32)


def _bf(x):
    return x.astype(jnp.bfloat16).astype(jnp.float32)


def _mlp_head(h, mw1_ref, mb1_ref, w2_ref, mb2_ref):
    h1 = _bdot(h, mw1_ref[...])
    h1 = jnp.maximum(h1 + mb1_ref[...], 0.0)
    s = jnp.sum(_bf(h1) * _bf(w2_ref[...]), axis=1, keepdims=True)
    return s + mb2_ref[...][:, :1]


def _layer_fn(part_ref, cnt_ref, x_ref, wl_ref, bl_ref, wr_ref,
              mw1_ref, mb1_ref, w2_ref, mb2_ref, xo_ref, *outs,
              l2, mlp):
    p3 = part_ref[...]
    c3 = cnt_ref[...]
    cc = c3[0, :, :1] + c3[1, :, :1]
    mean = (p3[0] + p3[1]) / jnp.maximum(cc, 1.0)
    h = _bdot(mean, wl_ref[...]) + bl_ref[...] + _bdot(x_ref[...], wr_ref[...])
    h = jnp.maximum(h, 0.0)
    if l2:
        n = jnp.sqrt(jnp.sum(h * h, axis=1, keepdims=True))
        h = h / jnp.maximum(n, 1e-12)
    xo_ref[...] = h
    if mlp:
        outs[0][...] = _mlp_head(h, mw1_ref, mb1_ref, w2_ref, mb2_ref)


def _layer(part, cnt, x, wl, bl, wr, mw1, mb1, w2, mb2, *, l2, mlp):
    outs = [jax.ShapeDtypeStruct((N, D), jnp.float32)]
    out_specs = [pl.BlockSpec((_RB, D), lambda i: (i, 0))]
    if mlp:
        outs.append(jax.ShapeDtypeStruct((N, 1), jnp.float32))
        out_specs.append(pl.BlockSpec((_RB, 1), lambda i: (i, 0)))
    full = lambda shape: pl.BlockSpec(shape, lambda i: tuple(0 for _ in shape))
    res = pl.pallas_call(
        functools.partial(_layer_fn, l2=l2, mlp=mlp),
        grid=(_GRID,),
        in_specs=[
            pl.BlockSpec((NC, _RB, D), lambda i: (0, i, 0)),
            pl.BlockSpec((NC, _RB, D), lambda i: (0, i, 0)),
            pl.BlockSpec((_RB, D), lambda i: (i, 0)),
            full((D, D)), full((1, D)), full((D, D)),
            full((D, D)), full((1, D)), full((1, D)), full((1, D)),
        ],
        out_specs=out_specs,
        out_shape=outs,
    )(part, cnt, x, wl, bl, wr, mw1, mb1, w2, mb2)
    if not mlp:
        return res[0], None
    return res


def _layer_nofuse_so_fn(x_ref, mw1_ref, mb1_ref, w2_ref, mb2_ref, so_ref):
    so_ref[...] = _mlp_head(x_ref[...], mw1_ref, mb1_ref, w2_ref, mb2_ref)


def _mlp_only(x, mw1, mb1, w2, mb2):
    full = lambda shape: pl.BlockSpec(shape, lambda i: tuple(0 for _ in shape))
    return pl.pallas_call(
        _layer_nofuse_so_fn,
        grid=(_GRID,),
        in_specs=[
            pl.BlockSpec((_RB, D), lambda i: (i, 0)),
            full((D, D)), full((1, D)), full((1, D)), full((1, D)),
        ],
        out_specs=pl.BlockSpec((_RB, 1), lambda i: (i, 0)),
        out_shape=jax.ShapeDtypeStruct((N, 1), jnp.float32),
    )(x, mw1, mb1, w2, mb2)


def _sum_fn(p_ref, o_ref):
    o_ref[...] = jnp.sum(p_ref[...], axis=1, keepdims=True)


def _sum_scores(parts):
    p = jnp.concatenate(parts, axis=1)
    k = p.shape[1]
    return pl.pallas_call(
        _sum_fn,
        grid=(_GRID,),
        in_specs=[pl.BlockSpec((_RB, k), lambda i: (i, 0))],
        out_specs=pl.BlockSpec((_RB, 1), lambda i: (i, 0)),
        out_shape=jax.ShapeDtypeStruct((N, 1), jnp.float32),
    )(p)


def kernel(adj1, adj2, W0, b0, Wl, bl, Wr, Wl_last, bl_last, Wr_last,
           mW1, mb1, mW2, mb2):
    src1d = adj2[0]
    dst3d = adj2[1].reshape(NW, ROWS_PW, CH)
    zeros = jnp.zeros((NP, D), jnp.float32)
    w2 = mW2.reshape(1, D)
    mb1r = mb1.reshape(1, D)
    mb2r = jnp.broadcast_to(mb2.reshape(1, 1), (1, D))
    mhead = (mW1, mb1r, w2, mb2r)

    cnt = _segcnt(dst3d, zeros)
    x0 = _init_matmul(adj1, W0, b0)

    m0 = _segsum(x0, src1d, dst3d, zeros)

    score_parts = [_mlp_only(x0, *mhead)]
    x1 = None
    for i in range(Wl.shape[0]):
        xi, si = _layer(m0, cnt, x0, Wl[i], bl[i].reshape(1, D), Wr[i],
                        *mhead, l2=True, mlp=True)
        score_parts.append(si)
        if i == 0:
            x1 = xi

    x = x1
    for i in range(1, Wl.shape[0]):
        mi = _segsum(x, src1d, dst3d, zeros)
        x, _ = _layer(mi, cnt, x, Wl[i], bl[i].reshape(1, D), Wr[i],
                      *mhead, l2=True, mlp=False)

    m_last = _segsum(x, src1d, dst3d, zeros)
    _, s_last = _layer(m_last, cnt, x, Wl_last, bl_last.reshape(1, D),
                       Wr_last, *mhead, l2=False, mlp=True)
    score_parts.append(s_last)

    return _sum_scores(score_parts)


# speedup vs baseline: 7.8842x; 1.0017x over previous
import functools

import jax
import jax.numpy as jnp
from jax import lax
from jax.experimental import pallas as pl
from jax.experimental.pallas import tpu as pltpu
from jax.experimental.pallas import tpu_sc as plsc

N = 10000
D = 128
E = 320000

NC = 2
NS = 16
NW = NC * NS
CH = 80
EPW = E // NW
ROWS_PW = EPW // CH
NP = 10240
NPW = NP // NS

_mesh = plsc.VectorSubcoreMesh(core_axis_name="c", subcore_axis_name="s")


def _segsum_body(x_hbm, src_hbm, dst_hbm, z_hbm, out_hbm,
                 src_v, dst_v, rows0, rows1, acc_sh,
                 sem0, sem1, ssem0, ssem1):
    c = lax.axis_index("c")
    s = lax.axis_index("s")
    wid = s * NC + c

    pltpu.sync_copy(z_hbm.at[pl.ds(s * NPW, NPW)],
                    acc_sh.at[pl.ds(s * NPW, NPW)])
    pltpu.sync_copy(src_hbm.at[pl.ds(wid * EPW, EPW)], src_v)
    pltpu.sync_copy(dst_hbm.at[wid], dst_v)
    plsc.subcore_barrier()

    def gather(j, buf, sem):
        pltpu.async_copy(x_hbm.at[src_v.at[pl.ds(j * CH, CH)]], buf, sem)

    def drain_scatter(j, buf, gsem, ssem):
        pltpu.make_async_copy(x_hbm.at[src_v.at[pl.ds(j * CH, CH)]], buf,
                              gsem).wait()
        pltpu.async_copy(buf, acc_sh.at[dst_v.at[j]], ssem, add=True)
        pltpu.make_async_copy(buf, acc_sh.at[dst_v.at[j]], ssem).wait()

    gather(0, rows0, sem0)

    def body(j, carry):
        gather(2 * j + 1, rows1, sem1)
        drain_scatter(2 * j, rows0, sem0, ssem0)
        gather(2 * j + 2, rows0, sem0)
        drain_scatter(2 * j + 1, rows1, sem1, ssem1)
        return carry

    lax.fori_loop(0, (ROWS_PW - 1) // 2, body, 0, unroll=False)
    drain_scatter(ROWS_PW - 1, rows0, sem0, ssem0)

    plsc.subcore_barrier()
    pltpu.sync_copy(acc_sh.at[pl.ds(s * NPW, NPW)],
                    out_hbm.at[c, pl.ds(s * NPW, NPW)])


_segsum = pl.kernel(
    _segsum_body,
    out_type=jax.ShapeDtypeStruct((NC, NP, D), jnp.float32),
    mesh=_mesh,
    scratch_types=[
        pltpu.VMEM((EPW,), jnp.int32),
        pltpu.VMEM((ROWS_PW, CH), jnp.int32),
        pltpu.VMEM((CH, D), jnp.float32),
        pltpu.VMEM((CH, D), jnp.float32),
        pltpu.VMEM_SHARED((NP, D), jnp.float32),
        pltpu.SemaphoreType.DMA,
        pltpu.SemaphoreType.DMA,
        pltpu.SemaphoreType.DMA,
        pltpu.SemaphoreType.DMA,
    ],
)


def _segcnt_body(dst_hbm, z_hbm, out_hbm, dst_v, ones_v, acc_sh, sem):
    c = lax.axis_index("c")
    s = lax.axis_index("s")
    wid = s * NC + c

    pltpu.sync_copy(z_hbm.at[pl.ds(s * NPW, NPW)],
                    acc_sh.at[pl.ds(s * NPW, NPW)])
    pltpu.sync_copy(dst_hbm.at[wid], dst_v)
    one16 = jnp.full((16,), 1.0, jnp.float32)
    for r in range(CH):
        for k in range(D // 16):
            ones_v[r, pl.ds(k * 16, 16)] = one16
    plsc.subcore_barrier()

    def body(j, carry):
        pltpu.async_copy(ones_v, acc_sh.at[dst_v.at[j]], sem, add=True)
        return carry

    lax.fori_loop(0, ROWS_PW, body, 0, unroll=False)

    def drain(j, carry):
        pltpu.make_async_copy(ones_v, acc_sh.at[dst_v.at[j]], sem).wait()
        return carry

    lax.fori_loop(0, ROWS_PW, drain, 0, unroll=False)
    plsc.subcore_barrier()
    pltpu.sync_copy(acc_sh.at[pl.ds(s * NPW, NPW)],
                    out_hbm.at[c, pl.ds(s * NPW, NPW)])


_segcnt = pl.kernel(
    _segcnt_body,
    out_type=jax.ShapeDtypeStruct((NC, NP, D), jnp.float32),
    mesh=_mesh,
    scratch_types=[
        pltpu.VMEM((ROWS_PW, CH), jnp.int32),
        pltpu.VMEM((CH, D), jnp.float32),
        pltpu.VMEM_SHARED((NP, D), jnp.float32),
        pltpu.SemaphoreType.DMA,
    ],
)


_RB = 400
_GRID = N // _RB


def _init_fn(a_ref, w_ref, b_ref, o_ref):
    h = jnp.dot(a_ref[...], w_ref[...], preferred_element_type=jnp.float32)
    h = jnp.maximum(h + b_ref[...], 0.0)
    n = jnp.sqrt(jnp.sum(h * h, axis=1, keepdims=True))
    o_ref[...] = h / jnp.maximum(n, 1e-12)


def _init_matmul(adj1, W0, b0):
    rb = 200
    return pl.pallas_call(
        _init_fn,
        grid=(N // rb,),
        in_specs=[
            pl.BlockSpec((rb, N), lambda i: (i, 0)),
            pl.BlockSpec((N, D), lambda i: (0, 0)),
            pl.BlockSpec((1, D), lambda i: (0, 0)),
        ],
        out_specs=pl.BlockSpec((rb, D), lambda i: (i, 0)),
        out_shape=jax.ShapeDtypeStruct((N, D), jnp.float32),
    )(adj1, W0, b0.reshape(1, D))


def _bdot(a, b):
    return jnp.dot(a.astype(jnp.bfloat16), b.astype(jnp.bfloat16),
                   preferred_element_type=jnp.float32)


def _bf(x):
    return x.astype(jnp.bfloat16).astype(jnp.float32)


def _mlp_head(h, mw1_ref, mb1_ref, w2_ref, mb2_ref):
    h1 = _bdot(h, mw1_ref[...])
    h1 = jnp.maximum(h1 + mb1_ref[...], 0.0)
    s = jnp.sum(_bf(h1) * _bf(w2_ref[...]), axis=1, keepdims=True)
    return s + mb2_ref[...][:, :1]


def _layer_fn(part_ref, cnt_ref, x_ref, wl_ref, bl_ref, wr_ref,
              mw1_ref, mb1_ref, w2_ref, mb2_ref, xo_ref, *outs,
              l2, mlp):
    p3 = part_ref[...]
    c3 = cnt_ref[...]
    cc = c3[0, :, :1] + c3[1, :, :1]
    mean = (p3[0] + p3[1]) / jnp.maximum(cc, 1.0)
    h = _bdot(mean, wl_ref[...]) + bl_ref[...] + _bdot(x_ref[...], wr_ref[...])
    h = jnp.maximum(h, 0.0)
    if l2:
        n = jnp.sqrt(jnp.sum(h * h, axis=1, keepdims=True))
        h = h / jnp.maximum(n, 1e-12)
    xo_ref[...] = h
    if mlp:
        outs[0][...] = _mlp_head(h, mw1_ref, mb1_ref, w2_ref, mb2_ref)


def _layer(part, cnt, x, wl, bl, wr, mw1, mb1, w2, mb2, *, l2, mlp):
    outs = [jax.ShapeDtypeStruct((N, D), jnp.float32)]
    out_specs = [pl.BlockSpec((_RB, D), lambda i: (i, 0))]
    if mlp:
        outs.append(jax.ShapeDtypeStruct((N, 1), jnp.float32))
        out_specs.append(pl.BlockSpec((_RB, 1), lambda i: (i, 0)))
    full = lambda shape: pl.BlockSpec(shape, lambda i: tuple(0 for _ in shape))
    res = pl.pallas_call(
        functools.partial(_layer_fn, l2=l2, mlp=mlp),
        grid=(_GRID,),
        in_specs=[
            pl.BlockSpec((NC, _RB, D), lambda i: (0, i, 0)),
            pl.BlockSpec((NC, _RB, D), lambda i: (0, i, 0)),
            pl.BlockSpec((_RB, D), lambda i: (i, 0)),
            full((D, D)), full((1, D)), full((D, D)),
            full((D, D)), full((1, D)), full((1, D)), full((1, D)),
        ],
        out_specs=out_specs,
        out_shape=outs,
    )(part, cnt, x, wl, bl, wr, mw1, mb1, w2, mb2)
    if not mlp:
        return res[0], None
    return res


def _layer_nofuse_so_fn(x_ref, mw1_ref, mb1_ref, w2_ref, mb2_ref, so_ref):
    so_ref[...] = _mlp_head(x_ref[...], mw1_ref, mb1_ref, w2_ref, mb2_ref)


def _mlp_only(x, mw1, mb1, w2, mb2):
    full = lambda shape: pl.BlockSpec(shape, lambda i: tuple(0 for _ in shape))
    return pl.pallas_call(
        _layer_nofuse_so_fn,
        grid=(_GRID,),
        in_specs=[
            pl.BlockSpec((_RB, D), lambda i: (i, 0)),
            full((D, D)), full((1, D)), full((1, D)), full((1, D)),
        ],
        out_specs=pl.BlockSpec((_RB, 1), lambda i: (i, 0)),
        out_shape=jax.ShapeDtypeStruct((N, 1), jnp.float32),
    )(x, mw1, mb1, w2, mb2)


def _sum_fn(p_ref, o_ref):
    o_ref[...] = jnp.sum(p_ref[...], axis=1, keepdims=True)


def _sum_scores(parts):
    p = jnp.concatenate(parts, axis=1)
    k = p.shape[1]
    return pl.pallas_call(
        _sum_fn,
        grid=(_GRID,),
        in_specs=[pl.BlockSpec((_RB, k), lambda i: (i, 0))],
        out_specs=pl.BlockSpec((_RB, 1), lambda i: (i, 0)),
        out_shape=jax.ShapeDtypeStruct((N, 1), jnp.float32),
    )(p)


def kernel(adj1, adj2, W0, b0, Wl, bl, Wr, Wl_last, bl_last, Wr_last,
           mW1, mb1, mW2, mb2):
    src1d = adj2[0]
    dst3d = adj2[1].reshape(NW, ROWS_PW, CH)
    zeros = jnp.zeros((NP, D), jnp.float32)
    w2 = mW2.reshape(1, D)
    mb1r = mb1.reshape(1, D)
    mb2r = jnp.broadcast_to(mb2.reshape(1, 1), (1, D))
    mhead = (mW1, mb1r, w2, mb2r)

    cnt = _segcnt(dst3d, zeros)
    x0 = _init_matmul(adj1, W0, b0)

    m0 = _segsum(x0, src1d, dst3d, zeros)

    score_parts = [_mlp_only(x0, *mhead)]
    x1 = None
    for i in range(Wl.shape[0]):
        xi, si = _layer(m0, cnt, x0, Wl[i], bl[i].reshape(1, D), Wr[i],
                        *mhead, l2=True, mlp=True)
        score_parts.append(si)
        if i == 0:
            x1 = xi

    x = x1
    for i in range(1, Wl.shape[0]):
        mi = _segsum(x, src1d, dst3d, zeros)
        x, _ = _layer(mi, cnt, x, Wl[i], bl[i].reshape(1, D), Wr[i],
                      *mhead, l2=True, mlp=False)

    m_last = _segsum(x, src1d, dst3d, zeros)
    _, s_last = _layer(m_last, cnt, x, Wl_last, bl_last.reshape(1, D),
                       Wr_last, *mhead, l2=False, mlp=True)
    score_parts.append(s_last)

    return _sum_scores(score_parts)
